# baseline (speedup 1.0000x reference)
import functools

import jax
import jax.numpy as jnp
from jax import lax
from jax.experimental import pallas as pl
from jax.experimental.pallas import tpu as pltpu
from jax.experimental.pallas import tpu_sc as plsc

N_C = 2
N_S = 16
N_W = N_C * N_S
CH = 128
NBUF = 10
SLK = 3


def _cdiv(a, b):
    return (a + b - 1) // b


@functools.lru_cache(maxsize=None)
def _sc_degree(n_pad, nchunk):
    mesh = plsc.VectorSubcoreMesh(core_axis_name="c", subcore_axis_name="s")
    rp = n_pad // N_S

    @functools.partial(
        pl.kernel,
        out_type=jax.ShapeDtypeStruct((N_C, n_pad, 16), jnp.float32),
        mesh=mesh,
        scratch_types=[
            pltpu.VMEM((nchunk, CH), jnp.int32),
            pltpu.VMEM((CH, 16), jnp.float32),
            pltpu.VMEM_SHARED((n_pad, 16), jnp.float32),
            pltpu.SemaphoreType.DMA,
        ],
        compiler_params=pltpu.CompilerParams(use_tc_tiling_on_sc=False),
    )
    def deg_kernel(dst_hbm, zeros_hbm, ones_hbm, out_hbm, idx_v, ones_v, acc,
                   ssem):
        cid = lax.axis_index("c")
        sid = lax.axis_index("s")
        wid = sid * N_C + cid
        row0 = sid * rp
        pltpu.sync_copy(zeros_hbm.at[pl.ds(row0, rp)], acc.at[pl.ds(row0, rp)])
        pltpu.sync_copy(ones_hbm, ones_v)
        pltpu.sync_copy(dst_hbm.at[wid], idx_v)
        plsc.subcore_barrier()

        def fire(j, carry):
            pltpu.async_copy(ones_v, acc.at[idx_v.at[j]], ssem, add=True)
            return carry

        lax.fori_loop(0, nchunk, fire, 0)

        def drain(j, carry):
            pltpu.make_async_copy(ones_v, acc.at[idx_v.at[0]], ssem).wait()
            return carry

        lax.fori_loop(0, nchunk, drain, 0)
        plsc.subcore_barrier()
        pltpu.sync_copy(acc.at[pl.ds(row0, rp)],
                        out_hbm.at[cid, pl.ds(row0, rp)])

    return deg_kernel


@functools.lru_cache(maxsize=None)
def _sc_aggregate(n_pad, nchunk, d):
    mesh = plsc.VectorSubcoreMesh(core_axis_name="c", subcore_axis_name="s")
    rp = n_pad // N_S

    @functools.partial(
        pl.kernel,
        out_type=jax.ShapeDtypeStruct((N_C, n_pad, d), jnp.float32),
        mesh=mesh,
        scratch_types=[
            pltpu.VMEM((nchunk, CH), jnp.int32),
            pltpu.VMEM((nchunk, CH), jnp.int32),
            pltpu.VMEM((NBUF, CH, d), jnp.float32),
            pltpu.VMEM_SHARED((n_pad, d), jnp.float32),
            pltpu.VMEM_SHARED((n_pad, d), jnp.float32),
            pltpu.SemaphoreType.DMA,
            pltpu.SemaphoreType.DMA,
        ],
        compiler_params=pltpu.CompilerParams(use_tc_tiling_on_sc=False),
    )
    def agg_kernel(h_hbm, src_hbm, dst_hbm, zeros_hbm, out_hbm,
                   src_v, dst_v, buf, acc, hsh, gsem, ssem):
        cid = lax.axis_index("c")
        sid = lax.axis_index("s")
        wid = sid * N_C + cid
        row0 = sid * rp
        pltpu.sync_copy(zeros_hbm.at[pl.ds(row0, rp)], acc.at[pl.ds(row0, rp)])
        pltpu.sync_copy(h_hbm.at[pl.ds(row0, rp)], hsh.at[pl.ds(row0, rp)])
        pltpu.sync_copy(src_hbm.at[wid], src_v)
        pltpu.sync_copy(dst_hbm.at[wid], dst_v)
        plsc.subcore_barrier()

        for b in range(min(NBUF, nchunk)):
            pltpu.async_copy(hsh.at[src_v.at[b]], buf.at[b], gsem)

        def body(j, carry):
            @pl.when(jnp.logical_and(j >= SLK, j + NBUF - SLK < nchunk))
            def _():
                pltpu.make_async_copy(
                    buf.at[0], acc.at[dst_v.at[0]], ssem).wait()
                nxt = j + NBUF - SLK
                pltpu.async_copy(hsh.at[src_v.at[nxt]],
                                 buf.at[lax.rem(nxt, NBUF)], gsem)

            slot = lax.rem(j, NBUF)
            pltpu.make_async_copy(
                hsh.at[src_v.at[j]], buf.at[slot], gsem).wait()
            pltpu.async_copy(buf.at[slot], acc.at[dst_v.at[j]], ssem, add=True)
            return carry

        lax.fori_loop(0, nchunk, body, 0)
        for _ in range(min(NBUF, nchunk)):
            pltpu.make_async_copy(
                buf.at[0], acc.at[dst_v.at[0]], ssem).wait()
        plsc.subcore_barrier()
        pltpu.sync_copy(acc.at[pl.ds(row0, rp)],
                        out_hbm.at[cid, pl.ds(row0, rp)])

    return agg_kernel


def _tc_matmul(x, w1):
    def body(x_ref, w1_ref, h_ref):
        h_ref[...] = jnp.dot(x_ref[...], w1_ref[...],
                             preferred_element_type=jnp.float32)

    return pl.pallas_call(
        body,
        out_shape=jax.ShapeDtypeStruct((x.shape[0], w1.shape[1]),
                                       jnp.float32),
    )(x, w1)


def _tc_scale(hraw, degp, n, n_pad):
    def body(hraw_ref, degp_ref, h_ref, dinv_ref):
        dp = degp_ref[...]
        deg = dp[0, :n, 0:1] + dp[1, :n, 0:1] + 1.0
        dinv = lax.rsqrt(deg)
        dinv_ref[...] = dinv
        h_ref[0:n, :] = hraw_ref[...] * dinv
        h_ref[n:n_pad, :] = jnp.zeros((n_pad - n, hraw_ref.shape[1]),
                                      jnp.float32)

    d = hraw.shape[1]
    return pl.pallas_call(
        body,
        out_shape=(jax.ShapeDtypeStruct((n_pad, d), jnp.float32),
                   jax.ShapeDtypeStruct((n, 1), jnp.float32)),
    )(hraw, degp)


def _tc_mid(aggp, h1, dinv, b1, n):
    def body(aggp_ref, h1_ref, dinv_ref, b1_ref, out_ref):
        ap = aggp_ref[...]
        n_pad = ap.shape[1]
        a = ap[0, :n, :] + ap[1, :n, :] + h1_ref[0:n, :]
        dinv = dinv_ref[...]
        out_ref[0:n, :] = jnp.maximum(a * dinv + b1_ref[...], 0.0) * dinv
        out_ref[n:n_pad, :] = jnp.zeros((n_pad - n, a.shape[1]), jnp.float32)

    d = h1.shape[1]
    return pl.pallas_call(
        body,
        out_shape=jax.ShapeDtypeStruct(aggp.shape[1:], jnp.float32),
    )(aggp, h1, dinv, b1)


def _tc_last(aggp, q, dinv, w2, b2, n):
    def body(aggp_ref, q_ref, dinv_ref, w2_ref, b2_ref, out_ref):
        ap = aggp_ref[...]
        g = (ap[0, :n, :] + ap[1, :n, :] + q_ref[0:n, :]) * dinv_ref[...]
        z = jnp.dot(g, w2_ref[...],
                    preferred_element_type=jnp.float32) + b2_ref[...]
        m = jnp.max(z, axis=1, keepdims=True)
        lse = jnp.log(jnp.sum(jnp.exp(z - m), axis=1, keepdims=True)) + m
        out_ref[...] = z - lse

    d = w2.shape[1]
    return pl.pallas_call(
        body,
        out_shape=jax.ShapeDtypeStruct((n, d), jnp.float32),
    )(aggp, q, dinv, w2, b2)


def kernel(x, edge_index, W1, b1, W2, b2):
    n = x.shape[0]
    e = edge_index.shape[1]
    d1 = W1.shape[1]
    d2 = W2.shape[1]
    n_pad = _cdiv(n + 1, N_S * 8) * N_S * 8
    nchunk = _cdiv(_cdiv(e, N_W), CH)
    e_pad = N_W * nchunk * CH

    ei = edge_index.astype(jnp.int32)
    src = jnp.concatenate(
        [ei[0], jnp.zeros((e_pad - e,), jnp.int32)]).reshape(N_W, nchunk, CH)
    dst = jnp.concatenate(
        [ei[1], jnp.full((e_pad - e,), n, jnp.int32)]).reshape(N_W, nchunk, CH)

    zeros1 = jnp.zeros((n_pad, d1), jnp.float32)
    ones_ch = jnp.ones((CH, 16), jnp.float32)

    degp = _sc_degree(n_pad, nchunk)(dst, jnp.zeros((n_pad, 16), jnp.float32),
                                     ones_ch)
    hraw = _tc_matmul(x, W1)
    h1, dinv = _tc_scale(hraw, degp, n, n_pad)
    agg1 = _sc_aggregate(n_pad, nchunk, d1)(h1, src, dst, zeros1)
    q = _tc_mid(agg1, h1, dinv, b1.reshape(1, d1), n)
    agg2 = _sc_aggregate(n_pad, nchunk, d1)(q, src, dst, zeros1)
    return _tc_last(agg2, q, dinv, W2, b2.reshape(1, d2), n)


# speedup vs baseline: 1.0351x; 1.0351x over previous
import functools

import jax
import jax.numpy as jnp
from jax import lax
from jax.experimental import pallas as pl
from jax.experimental.pallas import tpu as pltpu
from jax.experimental.pallas import tpu_sc as plsc

N_C = 2
N_S = 16
N_W = N_C * N_S
CH = 128
NBUF = 10
SLK = 3


def _cdiv(a, b):
    return (a + b - 1) // b


@functools.lru_cache(maxsize=None)
def _sc_degree(n_pad, nchunk):
    mesh = plsc.VectorSubcoreMesh(core_axis_name="c", subcore_axis_name="s")
    rp = n_pad // N_S

    @functools.partial(
        pl.kernel,
        out_type=jax.ShapeDtypeStruct((N_C, n_pad, 16), jnp.float32),
        mesh=mesh,
        scratch_types=[
            pltpu.VMEM((nchunk, CH), jnp.int32),
            pltpu.VMEM((CH, 16), jnp.float32),
            pltpu.VMEM_SHARED((n_pad, 16), jnp.float32),
            pltpu.SemaphoreType.DMA,
        ],
        compiler_params=pltpu.CompilerParams(use_tc_tiling_on_sc=False),
    )
    def deg_kernel(dst_hbm, zeros_hbm, ones_hbm, out_hbm, idx_v, ones_v, acc,
                   ssem):
        cid = lax.axis_index("c")
        sid = lax.axis_index("s")
        wid = sid * N_C + cid
        row0 = sid * rp
        pltpu.sync_copy(zeros_hbm.at[pl.ds(row0, rp)], acc.at[pl.ds(row0, rp)])
        pltpu.sync_copy(ones_hbm, ones_v)
        pltpu.sync_copy(dst_hbm.at[wid], idx_v)
        plsc.subcore_barrier()

        def fire(j, carry):
            pltpu.async_copy(ones_v, acc.at[idx_v.at[j]], ssem, add=True)
            return carry

        lax.fori_loop(0, nchunk, fire, 0)

        def drain(j, carry):
            pltpu.make_async_copy(ones_v, acc.at[idx_v.at[0]], ssem).wait()
            return carry

        lax.fori_loop(0, nchunk, drain, 0)
        plsc.subcore_barrier()
        pltpu.sync_copy(acc.at[pl.ds(row0, rp)],
                        out_hbm.at[cid, pl.ds(row0, rp)])

    return deg_kernel


@functools.lru_cache(maxsize=None)
def _sc_aggregate(n_pad, nchunk, d):
    mesh = plsc.VectorSubcoreMesh(core_axis_name="c", subcore_axis_name="s")
    rp = n_pad // N_S

    @functools.partial(
        pl.kernel,
        out_type=jax.ShapeDtypeStruct((N_C, n_pad, d), jnp.float32),
        mesh=mesh,
        scratch_types=[
            pltpu.VMEM((nchunk, CH), jnp.int32),
            pltpu.VMEM((nchunk, CH), jnp.int32),
            pltpu.VMEM((NBUF, CH, d), jnp.float32),
            pltpu.VMEM_SHARED((n_pad, d), jnp.float32),
            pltpu.VMEM_SHARED((n_pad, d), jnp.float32),
            pltpu.SemaphoreType.DMA,
            pltpu.SemaphoreType.DMA,
        ],
        compiler_params=pltpu.CompilerParams(use_tc_tiling_on_sc=False),
    )
    def agg_kernel(h_hbm, src_hbm, dst_hbm, zeros_hbm, out_hbm,
                   src_v, dst_v, buf, acc, hsh, gsem, ssem):
        cid = lax.axis_index("c")
        sid = lax.axis_index("s")
        wid = sid * N_C + cid
        row0 = sid * rp
        pltpu.async_copy(zeros_hbm.at[pl.ds(row0, rp)],
                         acc.at[pl.ds(row0, rp)], gsem)
        pltpu.async_copy(h_hbm.at[pl.ds(row0, rp)],
                         hsh.at[pl.ds(row0, rp)], gsem)
        pltpu.async_copy(src_hbm.at[wid], src_v, gsem)
        pltpu.async_copy(dst_hbm.at[wid], dst_v, gsem)
        pltpu.make_async_copy(zeros_hbm.at[pl.ds(row0, rp)],
                              acc.at[pl.ds(row0, rp)], gsem).wait()
        pltpu.make_async_copy(h_hbm.at[pl.ds(row0, rp)],
                              hsh.at[pl.ds(row0, rp)], gsem).wait()
        pltpu.make_async_copy(src_hbm.at[wid], src_v, gsem).wait()
        pltpu.make_async_copy(dst_hbm.at[wid], dst_v, gsem).wait()
        plsc.subcore_barrier()

        for b in range(min(NBUF, nchunk)):
            pltpu.async_copy(hsh.at[src_v.at[b]], buf.at[b], gsem)

        def body(j, carry):
            @pl.when(jnp.logical_and(j >= SLK, j + NBUF - SLK < nchunk))
            def _():
                pltpu.make_async_copy(
                    buf.at[0], acc.at[dst_v.at[0]], ssem).wait()
                nxt = j + NBUF - SLK
                pltpu.async_copy(hsh.at[src_v.at[nxt]],
                                 buf.at[lax.rem(nxt, NBUF)], gsem)

            slot = lax.rem(j, NBUF)
            pltpu.make_async_copy(
                hsh.at[src_v.at[j]], buf.at[slot], gsem).wait()
            pltpu.async_copy(buf.at[slot], acc.at[dst_v.at[j]], ssem, add=True)
            return carry

        lax.fori_loop(0, nchunk, body, 0)
        for _ in range(min(NBUF, nchunk)):
            pltpu.make_async_copy(
                buf.at[0], acc.at[dst_v.at[0]], ssem).wait()
        plsc.subcore_barrier()
        pltpu.sync_copy(acc.at[pl.ds(row0, rp)],
                        out_hbm.at[cid, pl.ds(row0, rp)])

    return agg_kernel


def _tc_first(x, degp, w1, n, n_pad):
    def body(x_ref, degp_ref, w1_ref, h_ref, dinv_ref):
        dp = degp_ref[...]
        deg = dp[0, :n, 0:1] + dp[1, :n, 0:1] + 1.0
        dinv = lax.rsqrt(deg)
        dinv_ref[...] = dinv
        h = jnp.dot(x_ref[...], w1_ref[...],
                    preferred_element_type=jnp.float32)
        h_ref[0:n, :] = h * dinv
        h_ref[n:n_pad, :] = jnp.zeros((n_pad - n, h.shape[1]), jnp.float32)

    d = w1.shape[1]
    return pl.pallas_call(
        body,
        out_shape=(jax.ShapeDtypeStruct((n_pad, d), jnp.float32),
                   jax.ShapeDtypeStruct((n, 1), jnp.float32)),
    )(x, degp, w1)


def _tc_mid(aggp, h1, dinv, b1, n):
    def body(aggp_ref, h1_ref, dinv_ref, b1_ref, out_ref):
        ap = aggp_ref[...]
        n_pad = ap.shape[1]
        a = ap[0, :n, :] + ap[1, :n, :] + h1_ref[0:n, :]
        dinv = dinv_ref[...]
        out_ref[0:n, :] = jnp.maximum(a * dinv + b1_ref[...], 0.0) * dinv
        out_ref[n:n_pad, :] = jnp.zeros((n_pad - n, a.shape[1]), jnp.float32)

    d = h1.shape[1]
    return pl.pallas_call(
        body,
        out_shape=jax.ShapeDtypeStruct(aggp.shape[1:], jnp.float32),
    )(aggp, h1, dinv, b1)


def _tc_last(aggp, q, dinv, w2, b2, n):
    def body(aggp_ref, q_ref, dinv_ref, w2_ref, b2_ref, out_ref):
        ap = aggp_ref[...]
        g = (ap[0, :n, :] + ap[1, :n, :] + q_ref[0:n, :]) * dinv_ref[...]
        z = jnp.dot(g, w2_ref[...],
                    preferred_element_type=jnp.float32) + b2_ref[...]
        m = jnp.max(z, axis=1, keepdims=True)
        lse = jnp.log(jnp.sum(jnp.exp(z - m), axis=1, keepdims=True)) + m
        out_ref[...] = z - lse

    d = w2.shape[1]
    return pl.pallas_call(
        body,
        out_shape=jax.ShapeDtypeStruct((n, d), jnp.float32),
    )(aggp, q, dinv, w2, b2)


def kernel(x, edge_index, W1, b1, W2, b2):
    n = x.shape[0]
    e = edge_index.shape[1]
    d1 = W1.shape[1]
    d2 = W2.shape[1]
    n_pad = _cdiv(n + 1, N_S * 8) * N_S * 8
    nchunk = _cdiv(_cdiv(e, N_W), CH)
    e_pad = N_W * nchunk * CH

    ei = edge_index.astype(jnp.int32)
    src = jnp.concatenate(
        [ei[0], jnp.zeros((e_pad - e,), jnp.int32)]).reshape(N_W, nchunk, CH)
    dst = jnp.concatenate(
        [ei[1], jnp.full((e_pad - e,), n, jnp.int32)]).reshape(N_W, nchunk, CH)

    zeros1 = jnp.zeros((n_pad, d1), jnp.float32)
    ones_ch = jnp.ones((CH, 16), jnp.float32)

    degp = _sc_degree(n_pad, nchunk)(dst, jnp.zeros((n_pad, 16), jnp.float32),
                                     ones_ch)
    h1, dinv = _tc_first(x, degp, W1, n, n_pad)
    agg1 = _sc_aggregate(n_pad, nchunk, d1)(h1, src, dst, zeros1)
    q = _tc_mid(agg1, h1, dinv, b1.reshape(1, d1), n)
    agg2 = _sc_aggregate(n_pad, nchunk, d1)(q, src, dst, zeros1)
    return _tc_last(agg2, q, dinv, W2, b2.reshape(1, d2), n)


# speedup vs baseline: 1.0358x; 1.0007x over previous
import functools

import jax
import jax.numpy as jnp
from jax import lax
from jax.experimental import pallas as pl
from jax.experimental.pallas import tpu as pltpu
from jax.experimental.pallas import tpu_sc as plsc

N_C = 2
N_S = 16
N_W = N_C * N_S
CH = 128
NBUF = 10
SLK = 3


def _cdiv(a, b):
    return (a + b - 1) // b


@functools.lru_cache(maxsize=None)
def _sc_degree(n_pad, nchunk):
    mesh = plsc.VectorSubcoreMesh(core_axis_name="c", subcore_axis_name="s")
    rp = n_pad // N_S

    @functools.partial(
        pl.kernel,
        out_type=jax.ShapeDtypeStruct((N_C, n_pad, 16), jnp.float32),
        mesh=mesh,
        scratch_types=[
            pltpu.VMEM((nchunk, CH), jnp.int32),
            pltpu.VMEM((CH, 16), jnp.float32),
            pltpu.VMEM_SHARED((n_pad, 16), jnp.float32),
            pltpu.SemaphoreType.DMA,
        ],
        compiler_params=pltpu.CompilerParams(use_tc_tiling_on_sc=False),
    )
    def deg_kernel(dst_hbm, zeros_hbm, ones_hbm, out_hbm, idx_v, ones_v, acc,
                   ssem):
        cid = lax.axis_index("c")
        sid = lax.axis_index("s")
        wid = sid * N_C + cid
        row0 = sid * rp
        pltpu.async_copy(zeros_hbm.at[pl.ds(row0, rp)],
                         acc.at[pl.ds(row0, rp)], ssem)
        pltpu.async_copy(ones_hbm, ones_v, ssem)
        pltpu.async_copy(dst_hbm.at[wid], idx_v, ssem)
        pltpu.make_async_copy(zeros_hbm.at[pl.ds(row0, rp)],
                              acc.at[pl.ds(row0, rp)], ssem).wait()
        pltpu.make_async_copy(ones_hbm, ones_v, ssem).wait()
        pltpu.make_async_copy(dst_hbm.at[wid], idx_v, ssem).wait()
        plsc.subcore_barrier()

        def fire(j, carry):
            pltpu.async_copy(ones_v, acc.at[idx_v.at[j]], ssem, add=True)
            return carry

        lax.fori_loop(0, nchunk, fire, 0)

        def drain(j, carry):
            pltpu.make_async_copy(ones_v, acc.at[idx_v.at[0]], ssem).wait()
            return carry

        lax.fori_loop(0, nchunk, drain, 0)
        plsc.subcore_barrier()
        pltpu.sync_copy(acc.at[pl.ds(row0, rp)],
                        out_hbm.at[cid, pl.ds(row0, rp)])

    return deg_kernel


@functools.lru_cache(maxsize=None)
def _sc_aggregate(n_pad, nchunk, d):
    mesh = plsc.VectorSubcoreMesh(core_axis_name="c", subcore_axis_name="s")
    rp = n_pad // N_S

    @functools.partial(
        pl.kernel,
        out_type=jax.ShapeDtypeStruct((N_C, n_pad, d), jnp.float32),
        mesh=mesh,
        scratch_types=[
            pltpu.VMEM((nchunk, CH), jnp.int32),
            pltpu.VMEM((nchunk, CH), jnp.int32),
            pltpu.VMEM((NBUF, CH, d), jnp.float32),
            pltpu.VMEM_SHARED((n_pad, d), jnp.float32),
            pltpu.VMEM_SHARED((n_pad, d), jnp.float32),
            pltpu.SemaphoreType.DMA,
            pltpu.SemaphoreType.DMA,
        ],
        compiler_params=pltpu.CompilerParams(use_tc_tiling_on_sc=False),
    )
    def agg_kernel(h_hbm, src_hbm, dst_hbm, zeros_hbm, out_hbm,
                   src_v, dst_v, buf, acc, hsh, gsem, ssem):
        cid = lax.axis_index("c")
        sid = lax.axis_index("s")
        wid = sid * N_C + cid
        row0 = sid * rp
        pltpu.async_copy(zeros_hbm.at[pl.ds(row0, rp)],
                         acc.at[pl.ds(row0, rp)], gsem)
        pltpu.async_copy(h_hbm.at[pl.ds(row0, rp)],
                         hsh.at[pl.ds(row0, rp)], gsem)
        pltpu.async_copy(src_hbm.at[wid], src_v, gsem)
        pltpu.async_copy(dst_hbm.at[wid], dst_v, gsem)
        pltpu.make_async_copy(zeros_hbm.at[pl.ds(row0, rp)],
                              acc.at[pl.ds(row0, rp)], gsem).wait()
        pltpu.make_async_copy(h_hbm.at[pl.ds(row0, rp)],
                              hsh.at[pl.ds(row0, rp)], gsem).wait()
        pltpu.make_async_copy(src_hbm.at[wid], src_v, gsem).wait()
        pltpu.make_async_copy(dst_hbm.at[wid], dst_v, gsem).wait()
        plsc.subcore_barrier()

        for b in range(min(NBUF, nchunk)):
            pltpu.async_copy(hsh.at[src_v.at[b]], buf.at[b], gsem)

        def body(j, carry):
            @pl.when(jnp.logical_and(j >= SLK, j + NBUF - SLK < nchunk))
            def _():
                pltpu.make_async_copy(
                    buf.at[0], acc.at[dst_v.at[0]], ssem).wait()
                nxt = j + NBUF - SLK
                pltpu.async_copy(hsh.at[src_v.at[nxt]],
                                 buf.at[lax.rem(nxt, NBUF)], gsem)

            slot = lax.rem(j, NBUF)
            pltpu.make_async_copy(
                hsh.at[src_v.at[j]], buf.at[slot], gsem).wait()
            pltpu.async_copy(buf.at[slot], acc.at[dst_v.at[j]], ssem, add=True)
            return carry

        lax.fori_loop(0, nchunk, body, 0)
        for _ in range(min(NBUF, nchunk)):
            pltpu.make_async_copy(
                buf.at[0], acc.at[dst_v.at[0]], ssem).wait()
        plsc.subcore_barrier()
        pltpu.sync_copy(acc.at[pl.ds(row0, rp)],
                        out_hbm.at[cid, pl.ds(row0, rp)])

    return agg_kernel


def _tc_first(x, degp, w1, n, n_pad):
    def body(x_ref, degp_ref, w1_ref, h_ref, dinv_ref):
        dp = degp_ref[...]
        deg = dp[0, :n, 0:1] + dp[1, :n, 0:1] + 1.0
        dinv = lax.rsqrt(deg)
        dinv_ref[...] = dinv
        h = jnp.dot(x_ref[...], w1_ref[...],
                    preferred_element_type=jnp.float32)
        h_ref[0:n, :] = h * dinv
        h_ref[n:n_pad, :] = jnp.zeros((n_pad - n, h.shape[1]), jnp.float32)

    d = w1.shape[1]
    return pl.pallas_call(
        body,
        out_shape=(jax.ShapeDtypeStruct((n_pad, d), jnp.float32),
                   jax.ShapeDtypeStruct((n, 1), jnp.float32)),
    )(x, degp, w1)


def _tc_mid(aggp, h1, dinv, b1, n):
    def body(aggp_ref, h1_ref, dinv_ref, b1_ref, out_ref):
        ap = aggp_ref[...]
        n_pad = ap.shape[1]
        a = ap[0, :n, :] + ap[1, :n, :] + h1_ref[0:n, :]
        dinv = dinv_ref[...]
        out_ref[0:n, :] = jnp.maximum(a * dinv + b1_ref[...], 0.0) * dinv
        out_ref[n:n_pad, :] = jnp.zeros((n_pad - n, a.shape[1]), jnp.float32)

    d = h1.shape[1]
    return pl.pallas_call(
        body,
        out_shape=jax.ShapeDtypeStruct(aggp.shape[1:], jnp.float32),
    )(aggp, h1, dinv, b1)


def _tc_last(aggp, q, dinv, w2, b2, n):
    def body(aggp_ref, q_ref, dinv_ref, w2_ref, b2_ref, out_ref):
        ap = aggp_ref[...]
        g = (ap[0, :n, :] + ap[1, :n, :] + q_ref[0:n, :]) * dinv_ref[...]
        z = jnp.dot(g, w2_ref[...],
                    preferred_element_type=jnp.float32) + b2_ref[...]
        m = jnp.max(z, axis=1, keepdims=True)
        lse = jnp.log(jnp.sum(jnp.exp(z - m), axis=1, keepdims=True)) + m
        out_ref[...] = z - lse

    d = w2.shape[1]
    return pl.pallas_call(
        body,
        out_shape=jax.ShapeDtypeStruct((n, d), jnp.float32),
    )(aggp, q, dinv, w2, b2)


def kernel(x, edge_index, W1, b1, W2, b2):
    n = x.shape[0]
    e = edge_index.shape[1]
    d1 = W1.shape[1]
    d2 = W2.shape[1]
    n_pad = _cdiv(n + 1, N_S * 8) * N_S * 8
    nchunk = _cdiv(_cdiv(e, N_W), CH)
    e_pad = N_W * nchunk * CH

    ei = edge_index.astype(jnp.int32)
    src = jnp.concatenate(
        [ei[0], jnp.zeros((e_pad - e,), jnp.int32)]).reshape(N_W, nchunk, CH)
    dst = jnp.concatenate(
        [ei[1], jnp.full((e_pad - e,), n, jnp.int32)]).reshape(N_W, nchunk, CH)

    zeros1 = jnp.zeros((n_pad, d1), jnp.float32)
    ones_ch = jnp.ones((CH, 16), jnp.float32)

    degp = _sc_degree(n_pad, nchunk)(dst, jnp.zeros((n_pad, 16), jnp.float32),
                                     ones_ch)
    h1, dinv = _tc_first(x, degp, W1, n, n_pad)
    agg1 = _sc_aggregate(n_pad, nchunk, d1)(h1, src, dst, zeros1)
    q = _tc_mid(agg1, h1, dinv, b1.reshape(1, d1), n)
    agg2 = _sc_aggregate(n_pad, nchunk, d1)(q, src, dst, zeros1)
    return _tc_last(agg2, q, dinv, W2, b2.reshape(1, d2), n)
